# baseline (speedup 1.0000x reference)
import jax
import jax.numpy as jnp
from jax import lax
from jax.experimental import pallas as pl
from jax.experimental.pallas import tpu as pltpu

_BETA = 64
_ALPHA = 2


def _lga_block_kernel(lc_ref, kxyz_ref, kx_ref, sph_ref, pct_ref, dir_ref, feat_ref):
    kxyz = kxyz_ref[...]
    kx = kx_ref[...]
    sph = sph_ref[...]
    T, K, _ = kxyz.shape
    C = kx.shape[2]

    lc = lc_ref[...]
    r = [kxyz[:, :, s:s + 1] - lc[:, s][:, None, None] for s in range(3)]
    dist = jnp.sqrt(r[0] * r[0] + r[1] * r[1] + r[2] * r[2])
    den = dist + 1e-08
    bf = lambda x: x.astype(jnp.bfloat16).astype(jnp.float32)
    n = [bf(r[s] / den) for s in range(3)]
    sphb = bf(sph)

    u = (n[0] * sphb[0, :][None, None, :]
         + n[1] * sphb[1, :][None, None, :]
         + n[2] * sphb[2, :][None, None, :])

    umax = jnp.max(u, axis=-1, keepdims=True)
    iota_a = lax.broadcasted_iota(jnp.int32, u.shape, 2)
    idx = jnp.min(jnp.where(u >= umax, iota_a, _BETA), axis=-1, keepdims=True)
    oh = (iota_a == idx).astype(jnp.float32)

    counts = jnp.sum(oh, axis=1)
    bin_iota = lax.broadcasted_iota(jnp.int32, counts.shape, 1)
    counts = counts - (bin_iota == 0).astype(jnp.float32)
    cden = counts + 1e-08

    for s in range(3):
        dir_ref[:, s, :] = jnp.sum(oh * bf(r[s]), axis=1) / cden

    iota_bins = lax.broadcasted_iota(jnp.int32, (T, _BETA, 1), 1)
    kxb = bf(kx)
    acc = jnp.zeros((T, _BETA, C), jnp.float32)
    for k in range(K):
        ohk = (iota_bins == idx[:, k, :][:, None, :]).astype(jnp.float32)
        acc = acc + ohk * kxb[:, k, :][:, None, :]
    fden = jnp.sum(acc, axis=-1, keepdims=True) + 1e-09
    feat_ref[...] = acc / fden

    thr = jnp.where(counts > _ALPHA, counts, 0.0)
    pct_ref[...] = thr / (jnp.sum(thr, axis=-1, keepdims=True) + 1e-08)


def kernel(lc_xyz, lc_x, knn_xyz, knn_x, sphere_points):
    B, N, K, C = knn_x.shape
    M = B * N
    T = 128

    lc = lc_xyz.reshape(M, 3)
    kxyz = knn_xyz.reshape(M, K, 3)
    kx = knn_x.reshape(M, K, C)
    sph = sphere_points.T

    pct, dirs, feat = pl.pallas_call(
        _lga_block_kernel,
        grid=(M // T,),
        in_specs=[
            pl.BlockSpec((T, 3), lambda i: (i, 0)),
            pl.BlockSpec((T, K, 3), lambda i: (i, 0, 0)),
            pl.BlockSpec((T, K, C), lambda i: (i, 0, 0)),
            pl.BlockSpec((3, _BETA), lambda i: (0, 0)),
        ],
        out_specs=[
            pl.BlockSpec((T, _BETA), lambda i: (i, 0)),
            pl.BlockSpec((T, 3, _BETA), lambda i: (i, 0, 0)),
            pl.BlockSpec((T, _BETA, C), lambda i: (i, 0, 0)),
        ],
        out_shape=[
            jax.ShapeDtypeStruct((M, _BETA), jnp.float32),
            jax.ShapeDtypeStruct((M, 3, _BETA), jnp.float32),
            jax.ShapeDtypeStruct((M, _BETA, C), jnp.float32),
        ],
        compiler_params=pltpu.CompilerParams(
            dimension_semantics=("arbitrary",),
        ),
    )(lc, kxyz, kx, sph)

    direction_percentage = pct.reshape(B, N, _BETA)
    avg_direction = jnp.transpose(dirs.reshape(B, N, 3, _BETA), (0, 1, 3, 2))
    avg_features = feat.reshape(B, N, _BETA, C)
    k_influence = jnp.ones((B, N), jnp.float32)
    return (knn_x, direction_percentage, avg_direction, avg_features, k_influence)


# speedup vs baseline: 1.4582x; 1.4582x over previous
import jax
import jax.numpy as jnp
from jax import lax
from jax.experimental import pallas as pl
from jax.experimental.pallas import tpu as pltpu

_BETA = 64
_ALPHA = 2
_G = 8


def _lga_block_kernel(kxyz_ref, lc_ref, kx_ref, sph_ref, pct_ref, dir_ref, feat_ref):
    kxyzr = kxyz_ref[...]
    lcr = lc_ref[...]
    kx = kx_ref[...]
    sph = sph_ref[...]
    R = kxyzr.shape[0]
    T = lcr.shape[0]
    K = R // T
    C = kx.shape[1]

    rel = kxyzr - jnp.broadcast_to(lcr[:, None, :], (T, K, 3)).reshape(R, 3)
    r0 = rel[:, 0:1]
    r1 = rel[:, 1:2]
    r2 = rel[:, 2:3]
    dist = jnp.sqrt(r0 * r0 + r1 * r1 + r2 * r2)
    den = dist + 1e-08
    nrm = (rel / den).astype(jnp.bfloat16)

    u = jax.lax.dot_general(
        nrm, sph.astype(jnp.bfloat16),
        (((1,), (0,)), ((), ())),
        preferred_element_type=jnp.float32,
    )

    umax = jnp.max(u, axis=-1, keepdims=True)
    lane_a = lax.broadcasted_iota(jnp.int32, (R, _BETA), 1)
    idx = jnp.min(jnp.where(u >= umax, lane_a, _BETA), axis=-1, keepdims=True)
    oh = (lane_a == idx).astype(jnp.float32)

    counts = jnp.sum(oh.reshape(T, K, _BETA), axis=1)
    bin_iota = lax.broadcasted_iota(jnp.int32, (T, _BETA), 1)
    counts = counts - (bin_iota == 0).astype(jnp.float32)
    thr = jnp.where(counts > _ALPHA, counts, 0.0)
    pct_ref[...] = thr / (jnp.sum(thr, axis=-1, keepdims=True) + 1e-08)

    xaug = jnp.concatenate(
        [kx.astype(jnp.bfloat16),
         rel.astype(jnp.bfloat16),
         jnp.ones((R, 1), jnp.bfloat16)],
        axis=1,
    )

    GK = _G * K
    GA = _G * _BETA
    col_iota = lax.broadcasted_iota(jnp.int32, (GK, GA), 1)
    row_off = (lax.broadcasted_iota(jnp.int32, (GK, 1), 0) // K) * _BETA

    ngroups = T // _G
    for g in range(ngroups):
        idx_g = idx[g * GK:(g + 1) * GK, :]
        ohbd = jnp.where(
            col_iota == row_off + idx_g, 1.0, 0.0
        ).astype(jnp.bfloat16)
        res = jax.lax.dot_general(
            ohbd, xaug[g * GK:(g + 1) * GK, :],
            (((0,), (0,)), ((), ())),
            preferred_element_type=jnp.float32,
        )
        feats = res[:, :C]
        dirs = res[:, C:C + 3]
        cnt = res[:, C + 3:C + 4]
        a_iota = lax.broadcasted_iota(jnp.int32, (GA, 1), 0) % _BETA
        cden = cnt - (a_iota == 0).astype(jnp.float32) + 1e-08
        fden = jnp.sum(feats, axis=-1, keepdims=True) + 1e-09
        feat_ref[g * GA:(g + 1) * GA, :] = feats / fden
        dir_ref[g * GA:(g + 1) * GA, :] = dirs / cden


def kernel(lc_xyz, lc_x, knn_xyz, knn_x, sphere_points):
    B, N, K, C = knn_x.shape
    M = B * N
    T = 128
    R = T * K

    kxyz = knn_xyz.reshape(M * K, 3)
    lc = lc_xyz.reshape(M, 3)
    kx = knn_x.reshape(M * K, C)
    sph = sphere_points.T

    pct, dirs, feat = pl.pallas_call(
        _lga_block_kernel,
        grid=(M // T,),
        in_specs=[
            pl.BlockSpec((R, 3), lambda i: (i, 0)),
            pl.BlockSpec((T, 3), lambda i: (i, 0)),
            pl.BlockSpec((R, C), lambda i: (i, 0)),
            pl.BlockSpec((3, _BETA), lambda i: (0, 0)),
        ],
        out_specs=[
            pl.BlockSpec((T, _BETA), lambda i: (i, 0)),
            pl.BlockSpec((T * _BETA, 3), lambda i: (i, 0)),
            pl.BlockSpec((T * _BETA, C), lambda i: (i, 0)),
        ],
        out_shape=[
            jax.ShapeDtypeStruct((M, _BETA), jnp.float32),
            jax.ShapeDtypeStruct((M * _BETA, 3), jnp.float32),
            jax.ShapeDtypeStruct((M * _BETA, C), jnp.float32),
        ],
        compiler_params=pltpu.CompilerParams(
            dimension_semantics=("arbitrary",),
        ),
    )(kxyz, lc, kx, sph)

    direction_percentage = pct.reshape(B, N, _BETA)
    avg_direction = dirs.reshape(B, N, _BETA, 3)
    avg_features = feat.reshape(B, N, _BETA, C)
    k_influence = jnp.ones((B, N), jnp.float32)
    return (knn_x, direction_percentage, avg_direction, avg_features, k_influence)


# speedup vs baseline: 2.1685x; 1.4871x over previous
import jax
import jax.numpy as jnp
from jax import lax
from jax.experimental import pallas as pl
from jax.experimental.pallas import tpu as pltpu

_BETA = 64
_ALPHA = 2
_G = 8


def _lga_block_kernel(kxyz_ref, lc_ref, kx_ref, sph_ref, pct_ref, dir_ref, feat_ref):
    kxyzr = kxyz_ref[...]
    lcr = lc_ref[...]
    kx = kx_ref[...]
    sph = sph_ref[...]
    R = kxyzr.shape[0]
    T = lcr.shape[0]
    K = R // T
    C = kx.shape[1]

    rel = kxyzr - jnp.broadcast_to(lcr[:, None, :], (T, K, 3)).reshape(R, 3)
    r0 = rel[:, 0:1]
    r1 = rel[:, 1:2]
    r2 = rel[:, 2:3]
    dist = jnp.sqrt(r0 * r0 + r1 * r1 + r2 * r2)
    den = dist + 1e-08
    nrm = (rel / den).astype(jnp.bfloat16)

    u = jax.lax.dot_general(
        nrm, sph.astype(jnp.bfloat16),
        (((1,), (0,)), ((), ())),
        preferred_element_type=jnp.float32,
    )

    umax = jnp.max(u, axis=-1, keepdims=True)
    lane_a = lax.broadcasted_iota(jnp.int32, (R, _BETA), 1)
    idx = jnp.min(jnp.where(u >= umax, lane_a, _BETA), axis=-1, keepdims=True)
    oh = (lane_a == idx).astype(jnp.float32)

    oh3 = oh.reshape(T, K, _BETA)
    counts = jnp.sum(oh3, axis=1)
    bin_iota = lax.broadcasted_iota(jnp.int32, (T, _BETA), 1)
    counts = counts - (bin_iota == 0).astype(jnp.float32)
    thr = jnp.where(counts > _ALPHA, counts, 0.0)
    pct_ref[...] = thr / (jnp.sum(thr, axis=-1, keepdims=True) + 1e-08)

    cden = counts + 1e-08
    relb = rel.astype(jnp.bfloat16).astype(jnp.float32)
    for s in range(3):
        rs3 = relb[:, s:s + 1].reshape(T, K, 1)
        dir_ref[:, s, :] = jnp.sum(oh3 * rs3, axis=1) / cden

    xaug = kx.astype(jnp.bfloat16)

    GK = _G * K
    GA = _G * _BETA
    col_iota = lax.broadcasted_iota(jnp.int32, (GK, GA), 1)
    row_off = (lax.broadcasted_iota(jnp.int32, (GK, 1), 0) // K) * _BETA

    ngroups = T // _G
    for g in range(ngroups):
        idx_g = idx[g * GK:(g + 1) * GK, :]
        ohbd = jnp.where(
            col_iota == row_off + idx_g, 1.0, 0.0
        ).astype(jnp.bfloat16)
        feats = jax.lax.dot_general(
            ohbd, xaug[g * GK:(g + 1) * GK, :],
            (((0,), (0,)), ((), ())),
            preferred_element_type=jnp.float32,
        )
        fden = jnp.sum(feats, axis=-1, keepdims=True) + 1e-09
        feat_ref[g * GA:(g + 1) * GA, :] = feats / fden


def kernel(lc_xyz, lc_x, knn_xyz, knn_x, sphere_points):
    B, N, K, C = knn_x.shape
    M = B * N
    T = 128
    R = T * K

    kxyz = knn_xyz.reshape(M * K, 3)
    lc = lc_xyz.reshape(M, 3)
    kx = knn_x.reshape(M * K, C)
    sph = sphere_points.T

    pct, dirs, feat = pl.pallas_call(
        _lga_block_kernel,
        grid=(M // T,),
        in_specs=[
            pl.BlockSpec((R, 3), lambda i: (i, 0)),
            pl.BlockSpec((T, 3), lambda i: (i, 0)),
            pl.BlockSpec((R, C), lambda i: (i, 0)),
            pl.BlockSpec((3, _BETA), lambda i: (0, 0)),
        ],
        out_specs=[
            pl.BlockSpec((T, _BETA), lambda i: (i, 0)),
            pl.BlockSpec((T, 3, _BETA), lambda i: (i, 0, 0)),
            pl.BlockSpec((T * _BETA, C), lambda i: (i, 0)),
        ],
        out_shape=[
            jax.ShapeDtypeStruct((M, _BETA), jnp.float32),
            jax.ShapeDtypeStruct((M, 3, _BETA), jnp.float32),
            jax.ShapeDtypeStruct((M * _BETA, C), jnp.float32),
        ],
        compiler_params=pltpu.CompilerParams(
            dimension_semantics=("arbitrary",),
        ),
    )(kxyz, lc, kx, sph)

    direction_percentage = pct.reshape(B, N, _BETA)
    avg_direction = jnp.transpose(dirs.reshape(B, N, 3, _BETA), (0, 1, 3, 2))
    avg_features = feat.reshape(B, N, _BETA, C)
    k_influence = jnp.ones((B, N), jnp.float32)
    return (knn_x, direction_percentage, avg_direction, avg_features, k_influence)
